# baseline (speedup 1.0000x reference)
import jax
import jax.numpy as jnp
from jax.experimental import pallas as pl
from jax.experimental.pallas import tpu as pltpu


def _memcpy_kernel(x_ref, o_ref, sem):
    copy = pltpu.make_async_copy(x_ref, o_ref, sem)
    copy.start()
    copy.wait()


def kernel(inputs, memories):
    del memories
    return pl.pallas_call(
        _memcpy_kernel,
        out_shape=jax.ShapeDtypeStruct(inputs.shape, inputs.dtype),
        in_specs=[pl.BlockSpec(memory_space=pl.ANY)],
        out_specs=pl.BlockSpec(memory_space=pl.ANY),
        scratch_shapes=[pltpu.SemaphoreType.DMA],
    )(inputs)


# speedup vs baseline: 1.0007x; 1.0007x over previous
import jax
import jax.numpy as jnp
from jax.experimental import pallas as pl
from jax.experimental.pallas import tpu as pltpu


_N_CHUNKS = 16


def _memcpy_kernel(x_ref, o_ref, sems):
    copies = [
        pltpu.make_async_copy(x_ref.at[i], o_ref.at[i], sems.at[i])
        for i in range(_N_CHUNKS)
    ]
    for c in copies:
        c.start()
    for c in copies:
        c.wait()


def kernel(inputs, memories):
    del memories
    B, T, d = inputs.shape
    x = inputs.reshape(_N_CHUNKS, (B * T) // _N_CHUNKS, d)
    out = pl.pallas_call(
        _memcpy_kernel,
        out_shape=jax.ShapeDtypeStruct(x.shape, x.dtype),
        in_specs=[pl.BlockSpec(memory_space=pl.ANY)],
        out_specs=pl.BlockSpec(memory_space=pl.ANY),
        scratch_shapes=[pltpu.SemaphoreType.DMA((_N_CHUNKS,))],
    )(x)
    return out.reshape(B, T, d)


# speedup vs baseline: 48.1597x; 48.1267x over previous
import jax
import jax.numpy as jnp
from jax.experimental import pallas as pl
from jax.experimental.pallas import tpu as pltpu


_BLOCK_ROWS = 512


def _memcpy_kernel(x_ref, o_ref):
    o_ref[...] = x_ref[...]


def kernel(inputs, memories):
    del memories
    B, T, d = inputs.shape
    x = inputs.reshape(B * T, d)
    grid = (B * T) // _BLOCK_ROWS
    out = pl.pallas_call(
        _memcpy_kernel,
        out_shape=jax.ShapeDtypeStruct(x.shape, x.dtype),
        grid=(grid,),
        in_specs=[pl.BlockSpec((_BLOCK_ROWS, d), lambda i: (i, 0))],
        out_specs=pl.BlockSpec((_BLOCK_ROWS, d), lambda i: (i, 0)),
    )(x)
    return out.reshape(B, T, d)


# speedup vs baseline: 49.1511x; 1.0206x over previous
import jax
import jax.numpy as jnp
from jax.experimental import pallas as pl
from jax.experimental.pallas import tpu as pltpu


_BLOCK_ROWS = 1024


def _memcpy_kernel(x_ref, o_ref):
    o_ref[...] = x_ref[...]


def kernel(inputs, memories):
    del memories
    B, T, d = inputs.shape
    x = inputs.reshape(B * T, d)
    grid = (B * T) // _BLOCK_ROWS
    out = pl.pallas_call(
        _memcpy_kernel,
        out_shape=jax.ShapeDtypeStruct(x.shape, x.dtype),
        grid=(grid,),
        in_specs=[pl.BlockSpec((_BLOCK_ROWS, d), lambda i: (i, 0))],
        out_specs=pl.BlockSpec((_BLOCK_ROWS, d), lambda i: (i, 0)),
    )(x)
    return out.reshape(B, T, d)
